# baseline (speedup 1.0000x reference)
import jax
import jax.numpy as jnp
from jax import lax
from jax.experimental import pallas as pl
from jax.experimental.pallas import tpu as pltpu
from jax.experimental.pallas import tpu_sc as plsc

N = 10000
E = 320000
D = 128
NE = 3

NC = 2
NS = 16
NW = NC * NS
C = 32
IB = 16
NBLK = 20
E_PAD = NW * NBLK * IB * C
STRIPE = 632
N_ACC = NS * STRIPE
TRASH = N
LAST = N - (NS - 1) * STRIPE


def _tables_body(x_ref, wk_ref, bk_ref, wq_ref, bq_ref, wv_ref, bv_ref,
                 ws_ref, b_ref, k_ref, qv_ref, skip_ref):
    xb = x_ref[...]
    dn = (((1,), (1,)), ((), ()))
    mm = lambda a, w: lax.dot_general(
        a, w, dn, precision=lax.Precision.HIGHEST,
        preferred_element_type=jnp.float32)
    for t in range(NE):
        k_ref[t] = mm(xb, wk_ref[t]) + bk_ref[t][None, :]
        qv_ref[t, :, 0:D] = mm(xb, wq_ref[t]) + bq_ref[t][None, :]
        qv_ref[t, :, D:2 * D] = mm(xb, wv_ref[t]) + bv_ref[t][None, :]
    ws_m = (ws_ref[0] + ws_ref[1] + ws_ref[2]) * (1.0 / NE)
    b_m = (b_ref[0] + b_ref[1] + b_ref[2]) * (1.0 / NE)
    skip_ref[...] = mm(xb, ws_m) + b_m[None, :]


def _make_tables(x, Wk, bk, Wq, bq, Wv, bv, Ws, b):
    bn = 1000
    full3 = pl.BlockSpec((NE, D, D), lambda i: (0, 0, 0))
    full2 = pl.BlockSpec((NE, D), lambda i: (0, 0))
    return pl.pallas_call(
        _tables_body,
        grid=(N // bn,),
        in_specs=[
            pl.BlockSpec((bn, D), lambda i: (i, 0)),
            full3, full2, full3, full2, full3, full2, full3, full2,
        ],
        out_specs=[
            pl.BlockSpec((NE, bn, D), lambda i: (0, i, 0)),
            pl.BlockSpec((NE, bn, 2 * D), lambda i: (0, i, 0)),
            pl.BlockSpec((bn, D), lambda i: (i, 0)),
        ],
        out_shape=[
            jax.ShapeDtypeStruct((NE, N, D), jnp.float32),
            jax.ShapeDtypeStruct((NE, N, 2 * D), jnp.float32),
            jax.ShapeDtypeStruct((N, D), jnp.float32),
        ],
    )(x, Wk, bk, Wq, bq, Wv, bv, Ws, b)


def _idx_body(src_ref, dst_ref, typ_ref, ik_ref, iqv_ref, id_ref):
    t = typ_ref[...]
    s = src_ref[...]
    d = dst_ref[...]
    valid = t < NE
    tn = t * N
    trash = TRASH + (lax.broadcasted_iota(jnp.int32, t.shape, 1) & 63)
    ik_ref[...] = jnp.where(valid, tn + d, 0)
    iqv_ref[...] = jnp.where(valid, tn + s, 0)
    id_ref[...] = jnp.where(valid, d, trash)


def _make_indices(src_p, dst_p, typ_p):
    rows = E_PAD // 128
    spec = pl.BlockSpec((8, 128), lambda i: (i, 0))
    ik, iqv, idst = pl.pallas_call(
        _idx_body,
        grid=(rows // 8,),
        in_specs=[spec, spec, spec],
        out_specs=[spec, spec, spec],
        out_shape=[jax.ShapeDtypeStruct((rows, 128), jnp.int32)] * 3,
    )(src_p.reshape(rows, 128), dst_p.reshape(rows, 128),
      typ_p.reshape(rows, 128))
    def shard(a):
        return a.reshape(NW, NBLK, IB, 1, C)
    return jnp.concatenate([shard(ik), shard(iqv), shard(idst)], axis=3)


def _edge_body(kt_hbm, qvt_hbm, idx_hbm, out_hbm,
               idx_v, rk0, rk1, rqv0, rqv1, msg0, msg1, acc,
               sk0, sk1, sq0, sq1, ss0, ss1):
    cid = lax.axis_index("c")
    sid = lax.axis_index("s")
    wid = sid * NC + cid
    rk = (rk0, rk1)
    rqv = (rqv0, rqv1)
    msg = (msg0, msg1)
    sk = (sk0, sk1)
    sq = (sq0, sq1)
    ss = (ss0, ss1)

    if True:
        zeros16 = jnp.zeros((16,), jnp.float32)

        @pl.loop(0, C)
        def _zrow(i):
            for g in range(8):
                msg0[i, pl.ds(g * 16, 16)] = zeros16

        lo = sid * STRIPE
        for r in range(STRIPE // C):
            pltpu.sync_copy(msg0, acc.at[pl.ds(lo + r * C, C)])
        rem = STRIPE % C
        if rem:
            pltpu.sync_copy(msg0.at[pl.ds(0, rem)],
                            acc.at[pl.ds(lo + (STRIPE // C) * C, rem)])
        plsc.subcore_barrier()

        def start_gather(j, s):
            pltpu.async_copy(kt_hbm.at[idx_v.at[j, 0]], rk[s], sk[s])
            pltpu.async_copy(qvt_hbm.at[idx_v.at[j, 1]], rqv[s], sq[s])

        def process(j, s, pf_j, pf_cond):
            pltpu.make_async_copy(kt_hbm.at[idx_v.at[j, 0]], rk[s],
                                  sk[s]).wait()
            pltpu.make_async_copy(qvt_hbm.at[idx_v.at[j, 1]], rqv[s],
                                  sq[s]).wait()

            @pl.when(pf_cond)
            def _pf():
                start_gather(pf_j, 1 - s)

            @pl.when(j >= 2)
            def _drain():
                pltpu.make_async_copy(msg[s], acc.at[idx_v.at[j, 2]],
                                      ss[s]).wait()

            @plsc.parallel_loop(0, C, unroll=2)
            def _erow(e):
                for g in range(8):
                    kk = rk[s][e, pl.ds(g * 16, 16)]
                    qq = rqv[s][e, pl.ds(g * 16, 16)]
                    vv = rqv[s][e, pl.ds(D + g * 16, 16)]
                    sg = 1.0 / (1.0 + jnp.exp(-(kk + qq)))
                    msg[s][e, pl.ds(g * 16, 16)] = sg * vv

            pltpu.async_copy(msg[s], acc.at[idx_v.at[j, 2]], ss[s], add=True)

        @pl.loop(0, NBLK)
        def _blk(bi):
            pltpu.sync_copy(idx_hbm.at[wid, bi], idx_v)
            start_gather(0, 0)

            @pl.loop(0, IB, step=2)
            def _pair(j):
                process(j, 0, j + 1, True)
                process(j + 1, 1, j + 2, j + 2 < IB)

            pltpu.make_async_copy(msg0, acc.at[idx_v.at[IB - 2, 2]],
                                  ss0).wait()
            pltpu.make_async_copy(msg1, acc.at[idx_v.at[IB - 1, 2]],
                                  ss1).wait()

        plsc.subcore_barrier()

        base = cid * N + lo
        pltpu.sync_copy(acc.at[pl.ds(lo, LAST)], out_hbm.at[pl.ds(base, LAST)])

        @pl.when(sid != NS - 1)
        def _tail():
            pltpu.sync_copy(acc.at[pl.ds(lo + LAST, STRIPE - LAST)],
                            out_hbm.at[pl.ds(base + LAST, STRIPE - LAST)])


def _edge_stage(k_tab, qv_tab, idx):
    mesh = plsc.VectorSubcoreMesh(core_axis_name="c", subcore_axis_name="s")
    run = pl.kernel(
        _edge_body,
        out_type=jax.ShapeDtypeStruct((NC * N, D), jnp.float32),
        mesh=mesh,
        scratch_types=[
            pltpu.VMEM((IB, 3, C), jnp.int32),
            pltpu.VMEM((C, D), jnp.float32),
            pltpu.VMEM((C, D), jnp.float32),
            pltpu.VMEM((C, 2 * D), jnp.float32),
            pltpu.VMEM((C, 2 * D), jnp.float32),
            pltpu.VMEM((C, D), jnp.float32),
            pltpu.VMEM((C, D), jnp.float32),
            pltpu.VMEM_SHARED((N_ACC, D), jnp.float32),
            pltpu.SemaphoreType.DMA,
            pltpu.SemaphoreType.DMA,
            pltpu.SemaphoreType.DMA,
            pltpu.SemaphoreType.DMA,
            pltpu.SemaphoreType.DMA,
            pltpu.SemaphoreType.DMA,
        ],
    )
    return run(k_tab, qv_tab, idx)


def _combine_body(p0_ref, p1_ref, skip_ref, out_ref):
    out_ref[...] = (p0_ref[...] + p1_ref[...]) * (1.0 / NE) + skip_ref[...]


def _combine(partial, skip):
    bn = 1000
    return pl.pallas_call(
        _combine_body,
        grid=(N // bn,),
        in_specs=[
            pl.BlockSpec((bn, D), lambda i: (i, 0)),
            pl.BlockSpec((bn, D), lambda i: (i + N // bn, 0)),
            pl.BlockSpec((bn, D), lambda i: (i, 0)),
        ],
        out_specs=pl.BlockSpec((bn, D), lambda i: (i, 0)),
        out_shape=jax.ShapeDtypeStruct((N, D), jnp.float32),
    )(partial, partial, skip)


def kernel(x, edge_index, edge_type, Wk, bk, Wq, bq, Wv, bv, Ws, b):
    assert x.shape == (N, D) and edge_index.shape == (2, E)

    k3, qv3, skip = _make_tables(x, Wk, bk, Wq, bq, Wv, bv, Ws, b)
    k_tab = k3.reshape(NE * N, D)
    qv_tab = qv3.reshape(NE * N, 2 * D)

    pad = E_PAD - E
    src_p = jnp.pad(edge_index[0], (0, pad))
    dst_p = jnp.pad(edge_index[1], (0, pad))
    typ_p = jnp.pad(edge_type, (0, pad), constant_values=NE)
    idx = _make_indices(src_p, dst_p, typ_p)

    partial = _edge_stage(k_tab, qv_tab, idx)
    return _combine(partial, skip)


# speedup vs baseline: 2.1162x; 2.1162x over previous
import jax
import jax.numpy as jnp
from jax import lax
from jax.experimental import pallas as pl
from jax.experimental.pallas import tpu as pltpu
from jax.experimental.pallas import tpu_sc as plsc

N = 10000
E = 320000
D = 128
NE = 3

NC = 2
NS = 16
NW = NC * NS
C = 32
IB = 16
BE = IB * C
EPW = 10240
NBLK = EPW // BE
EPL = E - (NW - 1) * EPW
NBLK_L = EPL // BE
STRIPE = 632
N_ACC = NS * STRIPE
LAST = N - (NS - 1) * STRIPE


def _tables_body(x_ref, wc_ref, bc_ref, k_ref, qv_ref, skip_ref):
    xb = x_ref[...]
    y = lax.dot_general(xb, wc_ref[...], (((1,), (0,)), ((), ())),
                        precision=lax.Precision.HIGHEST,
                        preferred_element_type=jnp.float32)
    for t in range(NE):
        k_ref[t] = y[:, t * D:(t + 1) * D] + bc_ref[t][None, :]
        qv_ref[t] = (y[:, 3 * D + t * 2 * D:3 * D + (t + 1) * 2 * D]
                     + jnp.concatenate([bc_ref[3 + 2 * t], bc_ref[4 + 2 * t]],
                                       axis=0)[None, :])
    ssum = (y[:, 9 * D:10 * D] + y[:, 10 * D:11 * D] + y[:, 11 * D:12 * D])
    bsum = bc_ref[9] + bc_ref[10] + bc_ref[11]
    skip_ref[...] = ssum * (1.0 / NE) + (bsum * (1.0 / NE))[None, :]


def _make_tables(x, Wk, bk, Wq, bq, Wv, bv, Ws, b):
    wkT = jnp.swapaxes(Wk, 1, 2)
    wqT = jnp.swapaxes(Wq, 1, 2)
    wvT = jnp.swapaxes(Wv, 1, 2)
    wsT = jnp.swapaxes(Ws, 1, 2)
    w_cat = jnp.concatenate(
        [wkT[0], wkT[1], wkT[2],
         wqT[0], wvT[0], wqT[1], wvT[1], wqT[2], wvT[2],
         wsT[0], wsT[1], wsT[2]], axis=1)
    b_cat = jnp.stack(
        [bk[0], bk[1], bk[2], bq[0], bv[0], bq[1], bv[1], bq[2], bv[2],
         b[0], b[1], b[2]], axis=0)
    bn = 1000
    return pl.pallas_call(
        _tables_body,
        grid=(N // bn,),
        in_specs=[
            pl.BlockSpec((bn, D), lambda i: (i, 0)),
            pl.BlockSpec((D, 12 * D), lambda i: (0, 0)),
            pl.BlockSpec((12, D), lambda i: (0, 0)),
        ],
        out_specs=[
            pl.BlockSpec((NE, bn, D), lambda i: (0, i, 0)),
            pl.BlockSpec((NE, bn, 2 * D), lambda i: (0, i, 0)),
            pl.BlockSpec((bn, D), lambda i: (i, 0)),
        ],
        out_shape=[
            jax.ShapeDtypeStruct((NE, N, D), jnp.float32),
            jax.ShapeDtypeStruct((NE, N, 2 * D), jnp.float32),
            jax.ShapeDtypeStruct((N, D), jnp.float32),
        ],
    )(x, w_cat, b_cat)


def _edge_body(kt_hbm, qvt_hbm, ei_hbm, et_hbm, out_hbm,
               src_v, dst_v, typ_v, ik_v, iqv_v, id_v,
               rk0, rk1, rqv0, rqv1, msg0, msg1, acc,
               sk0, sk1, sq0, sq1, ss0, ss1):
    cid = lax.axis_index("c")
    sid = lax.axis_index("s")
    wid = sid * NC + cid
    rk = (rk0, rk1)
    rqv = (rqv0, rqv1)
    msg = (msg0, msg1)
    sk = (sk0, sk1)
    sq = (sq0, sq1)
    ss = (ss0, ss1)

    zeros16 = jnp.zeros((16,), jnp.float32)

    @pl.loop(0, C)
    def _zrow(i):
        for g in range(8):
            msg0[i, pl.ds(g * 16, 16)] = zeros16

    lo = sid * STRIPE
    for r in range(STRIPE // C):
        pltpu.sync_copy(msg0, acc.at[pl.ds(lo + r * C, C)])
    rem = STRIPE % C
    if rem:
        pltpu.sync_copy(msg0.at[pl.ds(0, rem)],
                        acc.at[pl.ds(lo + (STRIPE // C) * C, rem)])
    plsc.subcore_barrier()

    def start_gather(j, s):
        pltpu.async_copy(kt_hbm.at[ik_v.at[j]], rk[s], sk[s])
        pltpu.async_copy(qvt_hbm.at[iqv_v.at[j]], rqv[s], sq[s])

    def process(j, s, pf_j, pf_cond):
        pltpu.make_async_copy(kt_hbm.at[ik_v.at[j]], rk[s], sk[s]).wait()
        pltpu.make_async_copy(qvt_hbm.at[iqv_v.at[j]], rqv[s], sq[s]).wait()

        @pl.when(pf_cond)
        def _pf():
            start_gather(pf_j, 1 - s)

        @pl.when(j >= 2)
        def _drain():
            pltpu.make_async_copy(msg[s], acc.at[id_v.at[j]], ss[s]).wait()

        @plsc.parallel_loop(0, C, unroll=2)
        def _erow(e):
            for g in range(8):
                kk = rk[s][e, pl.ds(g * 16, 16)]
                qq = rqv[s][e, pl.ds(g * 16, 16)]
                vv = rqv[s][e, pl.ds(D + g * 16, 16)]
                sg = 1.0 / (1.0 + jnp.exp(-(kk + qq)))
                msg[s][e, pl.ds(g * 16, 16)] = sg * vv

        pltpu.async_copy(msg[s], acc.at[id_v.at[j]], ss[s], add=True)

    nblk = jnp.where(wid == NW - 1, NBLK_L, NBLK)
    ebase = wid * EPW

    @pl.loop(0, nblk)
    def _blk(bi):
        eoff = ebase + bi * BE
        pltpu.sync_copy(ei_hbm.at[0, pl.ds(eoff, BE)], src_v)
        pltpu.sync_copy(ei_hbm.at[1, pl.ds(eoff, BE)], dst_v)
        pltpu.sync_copy(et_hbm.at[pl.ds(eoff, BE)], typ_v)

        @plsc.parallel_loop(0, IB, unroll=2)
        def _idx(j):
            for g in range(C // 16):
                sl = pl.ds(j * C + g * 16, 16)
                gl = pl.ds(g * 16, 16)
                tn = typ_v[sl] * N
                ik_v[j, gl] = tn + dst_v[sl]
                iqv_v[j, gl] = tn + src_v[sl]
                id_v[j, gl] = dst_v[sl]

        start_gather(0, 0)

        @pl.loop(0, IB, step=2)
        def _pair(j):
            process(j, 0, j + 1, True)
            process(j + 1, 1, j + 2, j + 2 < IB)

        pltpu.make_async_copy(msg0, acc.at[id_v.at[IB - 2]], ss0).wait()
        pltpu.make_async_copy(msg1, acc.at[id_v.at[IB - 1]], ss1).wait()

    plsc.subcore_barrier()

    base = cid * N + lo
    pltpu.sync_copy(acc.at[pl.ds(lo, LAST)], out_hbm.at[pl.ds(base, LAST)])

    @pl.when(sid != NS - 1)
    def _tail():
        pltpu.sync_copy(acc.at[pl.ds(lo + LAST, STRIPE - LAST)],
                        out_hbm.at[pl.ds(base + LAST, STRIPE - LAST)])


def _edge_stage(k_tab, qv_tab, edge_index, edge_type):
    mesh = plsc.VectorSubcoreMesh(core_axis_name="c", subcore_axis_name="s")
    run = pl.kernel(
        _edge_body,
        out_type=jax.ShapeDtypeStruct((NC * N, D), jnp.float32),
        mesh=mesh,
        scratch_types=[
            pltpu.VMEM((BE,), jnp.int32),
            pltpu.VMEM((BE,), jnp.int32),
            pltpu.VMEM((BE,), jnp.int32),
            pltpu.VMEM((IB, C), jnp.int32),
            pltpu.VMEM((IB, C), jnp.int32),
            pltpu.VMEM((IB, C), jnp.int32),
            pltpu.VMEM((C, D), jnp.float32),
            pltpu.VMEM((C, D), jnp.float32),
            pltpu.VMEM((C, 2 * D), jnp.float32),
            pltpu.VMEM((C, 2 * D), jnp.float32),
            pltpu.VMEM((C, D), jnp.float32),
            pltpu.VMEM((C, D), jnp.float32),
            pltpu.VMEM_SHARED((N_ACC, D), jnp.float32),
            pltpu.SemaphoreType.DMA,
            pltpu.SemaphoreType.DMA,
            pltpu.SemaphoreType.DMA,
            pltpu.SemaphoreType.DMA,
            pltpu.SemaphoreType.DMA,
            pltpu.SemaphoreType.DMA,
        ],
    )
    return run(k_tab, qv_tab, edge_index, edge_type)


def _combine_body(p0_ref, p1_ref, skip_ref, out_ref):
    out_ref[...] = (p0_ref[...] + p1_ref[...]) * (1.0 / NE) + skip_ref[...]


def _combine(partial, skip):
    bn = 1000
    return pl.pallas_call(
        _combine_body,
        grid=(N // bn,),
        in_specs=[
            pl.BlockSpec((bn, D), lambda i: (i, 0)),
            pl.BlockSpec((bn, D), lambda i: (i + N // bn, 0)),
            pl.BlockSpec((bn, D), lambda i: (i, 0)),
        ],
        out_specs=pl.BlockSpec((bn, D), lambda i: (i, 0)),
        out_shape=jax.ShapeDtypeStruct((N, D), jnp.float32),
    )(partial, partial, skip)


def kernel(x, edge_index, edge_type, Wk, bk, Wq, bq, Wv, bv, Ws, b):
    assert x.shape == (N, D) and edge_index.shape == (2, E)

    k3, qv3, skip = _make_tables(x, Wk, bk, Wq, bq, Wv, bv, Ws, b)
    k_tab = k3.reshape(NE * N, D)
    qv_tab = qv3.reshape(NE * N, 2 * D)

    partial = _edge_stage(k_tab, qv_tab, edge_index, edge_type)
    return _combine(partial, skip)


# speedup vs baseline: 2.4478x; 1.1567x over previous
import jax
import jax.numpy as jnp
from jax import lax
from jax.experimental import pallas as pl
from jax.experimental.pallas import tpu as pltpu
from jax.experimental.pallas import tpu_sc as plsc

N = 10000
E = 320000
D = 128
NE = 3

NC = 2
NS = 16
NW = NC * NS
C = 32
IB = 16
BE = IB * C
EPW = 10240
NBLK = EPW // BE
EPL = E - (NW - 1) * EPW
NBLK_L = EPL // BE
STRIPE = 632
N_ACC = NS * STRIPE
LAST = N - (NS - 1) * STRIPE


def _pack_pair(ya, yb):
    def rne_hi(v):
        u = pltpu.bitcast(v, jnp.int32)
        r = u + 0x7FFF + ((u >> 16) & 1)
        return r & jnp.int32(-65536)

    return lax.shift_right_logical(rne_hi(ya), 16) | rne_hi(yb)


def _tables_body(x_ref, wc_ref, bc_ref, k_ref, qv_ref, skip_ref):
    xb = x_ref[...]
    y = lax.dot_general(xb, wc_ref[...], (((1,), (0,)), ((), ())),
                        precision=lax.Precision.HIGHEST,
                        preferred_element_type=jnp.float32)
    for t in range(NE):
        k_ref[t] = y[:, t * D:(t + 1) * D] + bc_ref[t][None, :]
        qf = y[:, 3 * D + t * 2 * D:3 * D + t * 2 * D + D] + bc_ref[3 + 2 * t][None, :]
        vf = y[:, 3 * D + t * 2 * D + D:3 * D + (t + 1) * 2 * D] + bc_ref[4 + 2 * t][None, :]
        qv_ref[t] = _pack_pair(qf, vf)
    ssum = (y[:, 9 * D:10 * D] + y[:, 10 * D:11 * D] + y[:, 11 * D:12 * D])
    bsum = bc_ref[9] + bc_ref[10] + bc_ref[11]
    skip_ref[...] = ssum * (1.0 / NE) + (bsum * (1.0 / NE))[None, :]


def _make_tables(x, Wk, bk, Wq, bq, Wv, bv, Ws, b):
    wkT = jnp.swapaxes(Wk, 1, 2)
    wqT = jnp.swapaxes(Wq, 1, 2)
    wvT = jnp.swapaxes(Wv, 1, 2)
    wsT = jnp.swapaxes(Ws, 1, 2)
    w_cat = jnp.concatenate(
        [wkT[0], wkT[1], wkT[2],
         wqT[0], wvT[0], wqT[1], wvT[1], wqT[2], wvT[2],
         wsT[0], wsT[1], wsT[2]], axis=1)
    b_cat = jnp.stack(
        [bk[0], bk[1], bk[2], bq[0], bv[0], bq[1],
         bv[1], bq[2], bv[2], b[0], b[1], b[2]], axis=0)
    bn = 1000
    return pl.pallas_call(
        _tables_body,
        grid=(N // bn,),
        in_specs=[
            pl.BlockSpec((bn, D), lambda i: (i, 0)),
            pl.BlockSpec((D, 12 * D), lambda i: (0, 0)),
            pl.BlockSpec((12, D), lambda i: (0, 0)),
        ],
        out_specs=[
            pl.BlockSpec((NE, bn, D), lambda i: (0, i, 0)),
            pl.BlockSpec((NE, bn, D), lambda i: (0, i, 0)),
            pl.BlockSpec((bn, D), lambda i: (i, 0)),
        ],
        out_shape=[
            jax.ShapeDtypeStruct((NE, N, D), jnp.float32),
            jax.ShapeDtypeStruct((NE, N, D), jnp.int32),
            jax.ShapeDtypeStruct((N, D), jnp.float32),
        ],
    )(x, w_cat, b_cat)


def _edge_body(kt_hbm, qvt_hbm, ei_hbm, et_hbm, out_hbm,
               src_v, dst_v, typ_v, ik_v, iqv_v, id_v,
               rk0, rk1, rqv0, rqv1, msg0, msg1, acc,
               sk0, sk1, sq0, sq1, ss0, ss1):
    cid = lax.axis_index("c")
    sid = lax.axis_index("s")
    wid = sid * NC + cid
    rk = (rk0, rk1)
    rqv = (rqv0, rqv1)
    msg = (msg0, msg1)
    sk = (sk0, sk1)
    sq = (sq0, sq1)
    ss = (ss0, ss1)

    zeros16 = jnp.zeros((16,), jnp.float32)

    @pl.loop(0, C)
    def _zrow(i):
        for g in range(8):
            msg0[i, pl.ds(g * 16, 16)] = zeros16

    lo = sid * STRIPE
    for r in range(STRIPE // C):
        pltpu.sync_copy(msg0, acc.at[pl.ds(lo + r * C, C)])
    rem = STRIPE % C
    if rem:
        pltpu.sync_copy(msg0.at[pl.ds(0, rem)],
                        acc.at[pl.ds(lo + (STRIPE // C) * C, rem)])
    plsc.subcore_barrier()

    def start_gather(j, s):
        pltpu.async_copy(kt_hbm.at[ik_v.at[j]], rk[s], sk[s])
        pltpu.async_copy(qvt_hbm.at[iqv_v.at[j]], rqv[s], sq[s])

    def process(j, s, pf_j, pf_cond):
        pltpu.make_async_copy(kt_hbm.at[ik_v.at[j]], rk[s], sk[s]).wait()
        pltpu.make_async_copy(qvt_hbm.at[iqv_v.at[j]], rqv[s], sq[s]).wait()

        @pl.when(pf_cond)
        def _pf():
            start_gather(pf_j, 1 - s)

        @pl.when(j >= 2)
        def _drain():
            pltpu.make_async_copy(msg[s], acc.at[id_v.at[j]], ss[s]).wait()

        himask = jnp.full((16,), -65536, jnp.int32)

        def unpk(lanes):
            lof = lax.bitcast_convert_type(lanes << 16, jnp.float32)
            hif = lax.bitcast_convert_type(lanes & himask, jnp.float32)
            return lof, hif

        @plsc.parallel_loop(0, C, unroll=2)
        def _erow(e):
            for g in range(D // 32):
                ka = rk[s][e, pl.ds(g * 16, 16)]
                kb = rk[s][e, pl.ds(D // 2 + g * 16, 16)]
                q1, v1 = unpk(rqv[s][e, pl.ds(g * 16, 16)])
                q2, v2 = unpk(rqv[s][e, pl.ds(D // 2 + g * 16, 16)])
                s1 = 1.0 / (1.0 + jnp.exp(-(ka + q1)))
                s2 = 1.0 / (1.0 + jnp.exp(-(kb + q2)))
                msg[s][e, pl.ds(g * 16, 16)] = s1 * v1
                msg[s][e, pl.ds(D // 2 + g * 16, 16)] = s2 * v2

        pltpu.async_copy(msg[s], acc.at[id_v.at[j]], ss[s], add=True)

    nblk = jnp.where(wid == NW - 1, NBLK_L, NBLK)
    ebase = wid * EPW

    @pl.loop(0, nblk)
    def _blk(bi):
        eoff = ebase + bi * BE
        pltpu.sync_copy(ei_hbm.at[0, pl.ds(eoff, BE)], src_v)
        pltpu.sync_copy(ei_hbm.at[1, pl.ds(eoff, BE)], dst_v)
        pltpu.sync_copy(et_hbm.at[pl.ds(eoff, BE)], typ_v)

        @plsc.parallel_loop(0, IB, unroll=2)
        def _idx(j):
            for g in range(C // 16):
                sl = pl.ds(j * C + g * 16, 16)
                gl = pl.ds(g * 16, 16)
                tn = typ_v[sl] * N
                ik_v[j, gl] = tn + dst_v[sl]
                iqv_v[j, gl] = tn + src_v[sl]
                id_v[j, gl] = dst_v[sl]

        start_gather(0, 0)

        @pl.loop(0, IB, step=2)
        def _pair(j):
            process(j, 0, j + 1, True)
            process(j + 1, 1, j + 2, j + 2 < IB)

        pltpu.make_async_copy(msg0, acc.at[id_v.at[IB - 2]], ss0).wait()
        pltpu.make_async_copy(msg1, acc.at[id_v.at[IB - 1]], ss1).wait()

    plsc.subcore_barrier()

    base = cid * N + lo
    pltpu.sync_copy(acc.at[pl.ds(lo, LAST)], out_hbm.at[pl.ds(base, LAST)])

    @pl.when(sid != NS - 1)
    def _tail():
        pltpu.sync_copy(acc.at[pl.ds(lo + LAST, STRIPE - LAST)],
                        out_hbm.at[pl.ds(base + LAST, STRIPE - LAST)])


def _edge_stage(k_tab, qv_tab, edge_index, edge_type):
    mesh = plsc.VectorSubcoreMesh(core_axis_name="c", subcore_axis_name="s")
    run = pl.kernel(
        _edge_body,
        out_type=jax.ShapeDtypeStruct((NC * N, D), jnp.float32),
        mesh=mesh,
        scratch_types=[
            pltpu.VMEM((BE,), jnp.int32),
            pltpu.VMEM((BE,), jnp.int32),
            pltpu.VMEM((BE,), jnp.int32),
            pltpu.VMEM((IB, C), jnp.int32),
            pltpu.VMEM((IB, C), jnp.int32),
            pltpu.VMEM((IB, C), jnp.int32),
            pltpu.VMEM((C, D), jnp.float32),
            pltpu.VMEM((C, D), jnp.float32),
            pltpu.VMEM((C, D), jnp.int32),
            pltpu.VMEM((C, D), jnp.int32),
            pltpu.VMEM((C, D), jnp.float32),
            pltpu.VMEM((C, D), jnp.float32),
            pltpu.VMEM_SHARED((N_ACC, D), jnp.float32),
            pltpu.SemaphoreType.DMA,
            pltpu.SemaphoreType.DMA,
            pltpu.SemaphoreType.DMA,
            pltpu.SemaphoreType.DMA,
            pltpu.SemaphoreType.DMA,
            pltpu.SemaphoreType.DMA,
        ],
    )
    return run(k_tab, qv_tab, edge_index, edge_type)


def _combine_body(p0_ref, p1_ref, skip_ref, out_ref):
    out_ref[...] = (p0_ref[...] + p1_ref[...]) * (1.0 / NE) + skip_ref[...]


def _combine(partial, skip):
    bn = 1000
    return pl.pallas_call(
        _combine_body,
        grid=(N // bn,),
        in_specs=[
            pl.BlockSpec((bn, D), lambda i: (i, 0)),
            pl.BlockSpec((bn, D), lambda i: (i + N // bn, 0)),
            pl.BlockSpec((bn, D), lambda i: (i, 0)),
        ],
        out_specs=pl.BlockSpec((bn, D), lambda i: (i, 0)),
        out_shape=jax.ShapeDtypeStruct((N, D), jnp.float32),
    )(partial, partial, skip)


def kernel(x, edge_index, edge_type, Wk, bk, Wq, bq, Wv, bv, Ws, b):
    assert x.shape == (N, D) and edge_index.shape == (2, E)

    k3, qv3, skip = _make_tables(x, Wk, bk, Wq, bq, Wv, bv, Ws, b)
    k_tab = k3.reshape(NE * N, D)
    qv_tab = qv3.reshape(NE * N, D)

    partial = _edge_stage(k_tab, qv_tab, edge_index, edge_type)
    return _combine(partial, skip)


# speedup vs baseline: 2.9575x; 1.2082x over previous
import jax
import jax.numpy as jnp
from jax import lax
from jax.experimental import pallas as pl
from jax.experimental.pallas import tpu as pltpu
from jax.experimental.pallas import tpu_sc as plsc

N = 10000
E = 320000
D = 128
NE = 3

NC = 2
NS = 16
NW = NC * NS
C = 32
IB = 16
BE = IB * C
EPW = 10240
NBLK = EPW // BE
EPL = E - (NW - 1) * EPW
NBLK_L = EPL // BE
STRIPE = 632
N_ACC = NS * STRIPE
LAST = N - (NS - 1) * STRIPE


def _pack_pair(ya, yb):
    def rne_hi(v):
        u = pltpu.bitcast(v, jnp.int32)
        r = u + 0x7FFF + ((u >> 16) & 1)
        return r & jnp.int32(-65536)

    return lax.shift_right_logical(rne_hi(ya), 16) | rne_hi(yb)


def _tables_body(x_ref, wc_ref, bc_ref, k_ref, qv_ref, skip_ref):
    xb = x_ref[...]
    y = lax.dot_general(xb, wc_ref[...], (((1,), (0,)), ((), ())),
                        preferred_element_type=jnp.float32)
    for t in range(NE):
        k_ref[t] = y[:, t * D:(t + 1) * D] + bc_ref[t][None, :]
        qf = y[:, 3 * D + t * 2 * D:3 * D + t * 2 * D + D] + bc_ref[3 + 2 * t][None, :]
        vf = y[:, 3 * D + t * 2 * D + D:3 * D + (t + 1) * 2 * D] + bc_ref[4 + 2 * t][None, :]
        qv_ref[t] = _pack_pair(qf, vf)
    ssum = (y[:, 9 * D:10 * D] + y[:, 10 * D:11 * D] + y[:, 11 * D:12 * D])
    bsum = bc_ref[9] + bc_ref[10] + bc_ref[11]
    skip_ref[...] = ssum * (1.0 / NE) + (bsum * (1.0 / NE))[None, :]


def _make_tables(x, Wk, bk, Wq, bq, Wv, bv, Ws, b):
    wkT = jnp.swapaxes(Wk, 1, 2)
    wqT = jnp.swapaxes(Wq, 1, 2)
    wvT = jnp.swapaxes(Wv, 1, 2)
    wsT = jnp.swapaxes(Ws, 1, 2)
    w_cat = jnp.concatenate(
        [wkT[0], wkT[1], wkT[2],
         wqT[0], wvT[0], wqT[1], wvT[1], wqT[2], wvT[2],
         wsT[0], wsT[1], wsT[2]], axis=1)
    b_cat = jnp.stack(
        [bk[0], bk[1], bk[2], bq[0], bv[0], bq[1],
         bv[1], bq[2], bv[2], b[0], b[1], b[2]], axis=0)
    bn = 1000
    return pl.pallas_call(
        _tables_body,
        grid=(N // bn,),
        in_specs=[
            pl.BlockSpec((bn, D), lambda i: (i, 0)),
            pl.BlockSpec((D, 12 * D), lambda i: (0, 0)),
            pl.BlockSpec((12, D), lambda i: (0, 0)),
        ],
        out_specs=[
            pl.BlockSpec((NE, bn, D), lambda i: (0, i, 0)),
            pl.BlockSpec((NE, bn, D), lambda i: (0, i, 0)),
            pl.BlockSpec((bn, D), lambda i: (i, 0)),
        ],
        out_shape=[
            jax.ShapeDtypeStruct((NE, N, D), jnp.float32),
            jax.ShapeDtypeStruct((NE, N, D), jnp.int32),
            jax.ShapeDtypeStruct((N, D), jnp.float32),
        ],
    )(x, w_cat, b_cat)


def _edge_body(kt_hbm, qvt_hbm, ei_hbm, et_hbm, out_hbm,
               src_v, dst_v, typ_v, ik_v, iqv_v, id_v,
               rk0, rk1, rqv0, rqv1, msg0, msg1, acc,
               sk0, sk1, sq0, sq1, ss0, ss1):
    cid = lax.axis_index("c")
    sid = lax.axis_index("s")
    wid = sid * NC + cid
    rk = (rk0, rk1)
    rqv = (rqv0, rqv1)
    msg = (msg0, msg1)
    sk = (sk0, sk1)
    sq = (sq0, sq1)
    ss = (ss0, ss1)

    zeros16 = jnp.zeros((16,), jnp.float32)

    @pl.loop(0, C)
    def _zrow(i):
        for g in range(8):
            msg0[i, pl.ds(g * 16, 16)] = zeros16

    lo = sid * STRIPE
    for r in range(STRIPE // C):
        pltpu.sync_copy(msg0, acc.at[pl.ds(lo + r * C, C)])
    rem = STRIPE % C
    if rem:
        pltpu.sync_copy(msg0.at[pl.ds(0, rem)],
                        acc.at[pl.ds(lo + (STRIPE // C) * C, rem)])
    plsc.subcore_barrier()

    def start_gather(j, s):
        pltpu.async_copy(kt_hbm.at[ik_v.at[j]], rk[s], sk[s])
        pltpu.async_copy(qvt_hbm.at[iqv_v.at[j]], rqv[s], sq[s])

    def process(j, s, pf_j, pf_cond):
        @pl.when(pf_cond)
        def _pf():
            start_gather(pf_j, 1 - s)

        pltpu.make_async_copy(kt_hbm.at[ik_v.at[j]], rk[s], sk[s]).wait()
        pltpu.make_async_copy(qvt_hbm.at[iqv_v.at[j]], rqv[s], sq[s]).wait()

        @pl.when(j >= 2)
        def _drain():
            pltpu.make_async_copy(msg[s], acc.at[id_v.at[j]], ss[s]).wait()

        himask = jnp.full((16,), -65536, jnp.int32)

        def unpk(lanes):
            lof = lax.bitcast_convert_type(lanes << 16, jnp.float32)
            hif = lax.bitcast_convert_type(lanes & himask, jnp.float32)
            return lof, hif

        @plsc.parallel_loop(0, C, unroll=2)
        def _erow(e):
            for g in range(D // 32):
                ka = rk[s][e, pl.ds(g * 16, 16)]
                kb = rk[s][e, pl.ds(D // 2 + g * 16, 16)]
                q1, v1 = unpk(rqv[s][e, pl.ds(g * 16, 16)])
                q2, v2 = unpk(rqv[s][e, pl.ds(D // 2 + g * 16, 16)])
                s1 = 1.0 / (1.0 + jnp.exp(-(ka + q1)))
                s2 = 1.0 / (1.0 + jnp.exp(-(kb + q2)))
                msg[s][e, pl.ds(g * 16, 16)] = s1 * v1
                msg[s][e, pl.ds(D // 2 + g * 16, 16)] = s2 * v2

        pltpu.async_copy(msg[s], acc.at[id_v.at[j]], ss[s], add=True)

    nblk = jnp.where(wid == NW - 1, NBLK_L, NBLK)
    ebase = wid * EPW

    @pl.loop(0, nblk)
    def _blk(bi):
        eoff = ebase + bi * BE
        pltpu.sync_copy(ei_hbm.at[0, pl.ds(eoff, BE)], src_v)
        pltpu.sync_copy(ei_hbm.at[1, pl.ds(eoff, BE)], dst_v)
        pltpu.sync_copy(et_hbm.at[pl.ds(eoff, BE)], typ_v)

        @plsc.parallel_loop(0, IB, unroll=2)
        def _idx(j):
            for g in range(C // 16):
                sl = pl.ds(j * C + g * 16, 16)
                gl = pl.ds(g * 16, 16)
                tn = typ_v[sl] * N
                ik_v[j, gl] = tn + dst_v[sl]
                iqv_v[j, gl] = tn + src_v[sl]
                id_v[j, gl] = dst_v[sl]

        start_gather(0, 0)

        @pl.loop(0, IB, step=2)
        def _pair(j):
            process(j, 0, j + 1, True)
            process(j + 1, 1, j + 2, j + 2 < IB)

        pltpu.make_async_copy(msg0, acc.at[id_v.at[IB - 2]], ss0).wait()
        pltpu.make_async_copy(msg1, acc.at[id_v.at[IB - 1]], ss1).wait()

    plsc.subcore_barrier()

    base = cid * N + lo
    pltpu.sync_copy(acc.at[pl.ds(lo, LAST)], out_hbm.at[pl.ds(base, LAST)])

    @pl.when(sid != NS - 1)
    def _tail():
        pltpu.sync_copy(acc.at[pl.ds(lo + LAST, STRIPE - LAST)],
                        out_hbm.at[pl.ds(base + LAST, STRIPE - LAST)])


def _edge_stage(k_tab, qv_tab, edge_index, edge_type):
    mesh = plsc.VectorSubcoreMesh(core_axis_name="c", subcore_axis_name="s")
    run = pl.kernel(
        _edge_body,
        out_type=jax.ShapeDtypeStruct((NC * N, D), jnp.float32),
        mesh=mesh,
        scratch_types=[
            pltpu.VMEM((BE,), jnp.int32),
            pltpu.VMEM((BE,), jnp.int32),
            pltpu.VMEM((BE,), jnp.int32),
            pltpu.VMEM((IB, C), jnp.int32),
            pltpu.VMEM((IB, C), jnp.int32),
            pltpu.VMEM((IB, C), jnp.int32),
            pltpu.VMEM((C, D), jnp.float32),
            pltpu.VMEM((C, D), jnp.float32),
            pltpu.VMEM((C, D), jnp.int32),
            pltpu.VMEM((C, D), jnp.int32),
            pltpu.VMEM((C, D), jnp.float32),
            pltpu.VMEM((C, D), jnp.float32),
            pltpu.VMEM_SHARED((N_ACC, D), jnp.float32),
            pltpu.SemaphoreType.DMA,
            pltpu.SemaphoreType.DMA,
            pltpu.SemaphoreType.DMA,
            pltpu.SemaphoreType.DMA,
            pltpu.SemaphoreType.DMA,
            pltpu.SemaphoreType.DMA,
        ],
    )
    return run(k_tab, qv_tab, edge_index, edge_type)


def _combine_body(p0_ref, p1_ref, skip_ref, out_ref):
    out_ref[...] = (p0_ref[...] + p1_ref[...]) * (1.0 / NE) + skip_ref[...]


def _combine(partial, skip):
    bn = 1000
    return pl.pallas_call(
        _combine_body,
        grid=(N // bn,),
        in_specs=[
            pl.BlockSpec((bn, D), lambda i: (i, 0)),
            pl.BlockSpec((bn, D), lambda i: (i + N // bn, 0)),
            pl.BlockSpec((bn, D), lambda i: (i, 0)),
        ],
        out_specs=pl.BlockSpec((bn, D), lambda i: (i, 0)),
        out_shape=jax.ShapeDtypeStruct((N, D), jnp.float32),
    )(partial, partial, skip)


def kernel(x, edge_index, edge_type, Wk, bk, Wq, bq, Wv, bv, Ws, b):
    assert x.shape == (N, D) and edge_index.shape == (2, E)

    k3, qv3, skip = _make_tables(x, Wk, bk, Wq, bq, Wv, bv, Ws, b)
    k_tab = k3.reshape(NE * N, D)
    qv_tab = qv3.reshape(NE * N, D)

    partial = _edge_stage(k_tab, qv_tab, edge_index, edge_type)
    return _combine(partial, skip)


# speedup vs baseline: 2.9754x; 1.0061x over previous
import jax
import jax.numpy as jnp
from jax import lax
from jax.experimental import pallas as pl
from jax.experimental.pallas import tpu as pltpu
from jax.experimental.pallas import tpu_sc as plsc

N = 10000
E = 320000
D = 128
NE = 3

NC = 2
NS = 16
NW = NC * NS
C = 32
IB = 16
BE = IB * C
EPW = 10240
NBLK = EPW // BE
EPL = E - (NW - 1) * EPW
NBLK_L = EPL // BE
STRIPE = 632
N_ACC = NS * STRIPE
LAST = N - (NS - 1) * STRIPE


def _pack_pair(ya, yb):
    def rne_hi(v):
        u = pltpu.bitcast(v, jnp.int32)
        r = u + 0x7FFF + ((u >> 16) & 1)
        return r & jnp.int32(-65536)

    return lax.shift_right_logical(rne_hi(ya), 16) | rne_hi(yb)


def _tables_body(x_ref, wc_ref, bc_ref, k_ref, qv_ref, skip_ref):
    xb = x_ref[...]
    y = lax.dot_general(xb, wc_ref[...], (((1,), (0,)), ((), ())),
                        preferred_element_type=jnp.float32)
    for t in range(NE):
        k_ref[t] = y[:, t * D:(t + 1) * D] + bc_ref[t][None, :]
        qf = y[:, 3 * D + t * 2 * D:3 * D + t * 2 * D + D] + bc_ref[3 + 2 * t][None, :]
        vf = y[:, 3 * D + t * 2 * D + D:3 * D + (t + 1) * 2 * D] + bc_ref[4 + 2 * t][None, :]
        qv_ref[t] = _pack_pair(qf, vf)
    ssum = (y[:, 9 * D:10 * D] + y[:, 10 * D:11 * D] + y[:, 11 * D:12 * D])
    bsum = bc_ref[9] + bc_ref[10] + bc_ref[11]
    skip_ref[...] = ssum * (1.0 / NE) + (bsum * (1.0 / NE))[None, :]


def _make_tables(x, Wk, bk, Wq, bq, Wv, bv, Ws, b):
    wkT = jnp.swapaxes(Wk, 1, 2)
    wqT = jnp.swapaxes(Wq, 1, 2)
    wvT = jnp.swapaxes(Wv, 1, 2)
    wsT = jnp.swapaxes(Ws, 1, 2)
    w_cat = jnp.concatenate(
        [wkT[0], wkT[1], wkT[2],
         wqT[0], wvT[0], wqT[1], wvT[1], wqT[2], wvT[2],
         wsT[0], wsT[1], wsT[2]], axis=1)
    b_cat = jnp.stack(
        [bk[0], bk[1], bk[2], bq[0], bv[0], bq[1],
         bv[1], bq[2], bv[2], b[0], b[1], b[2]], axis=0)
    bn = 1000
    return pl.pallas_call(
        _tables_body,
        grid=(N // bn,),
        in_specs=[
            pl.BlockSpec((bn, D), lambda i: (i, 0)),
            pl.BlockSpec((D, 12 * D), lambda i: (0, 0)),
            pl.BlockSpec((12, D), lambda i: (0, 0)),
        ],
        out_specs=[
            pl.BlockSpec((NE, bn, D), lambda i: (0, i, 0)),
            pl.BlockSpec((NE, bn, D), lambda i: (0, i, 0)),
            pl.BlockSpec((bn, D), lambda i: (i, 0)),
        ],
        out_shape=[
            jax.ShapeDtypeStruct((NE, N, D), jnp.float32),
            jax.ShapeDtypeStruct((NE, N, D), jnp.int32),
            jax.ShapeDtypeStruct((N, D), jnp.float32),
        ],
    )(x, w_cat, b_cat)


def _edge_body(kt_hbm, qvt_hbm, ei_hbm, et_hbm, out_hbm,
               src_v, dst_v, typ_v, ik_v, iqv_v, id_v,
               rk0, rk1, rqv0, rqv1, msg0, msg1, acc,
               sk0, sk1, sq0, sq1, ss0, ss1):
    cid = lax.axis_index("c")
    sid = lax.axis_index("s")
    wid = sid * NC + cid
    rk = (rk0, rk1)
    rqv = (rqv0, rqv1)
    msg = (msg0, msg1)
    sk = (sk0, sk1)
    sq = (sq0, sq1)
    ss = (ss0, ss1)

    zeros16 = jnp.zeros((16,), jnp.float32)

    @pl.loop(0, C)
    def _zrow(i):
        for g in range(8):
            msg0[i, pl.ds(g * 16, 16)] = zeros16

    lo = sid * STRIPE
    for r in range(STRIPE // C):
        pltpu.sync_copy(msg0, acc.at[pl.ds(lo + r * C, C)])
    rem = STRIPE % C
    if rem:
        pltpu.sync_copy(msg0.at[pl.ds(0, rem)],
                        acc.at[pl.ds(lo + (STRIPE // C) * C, rem)])
    plsc.subcore_barrier()

    def start_gather(j, s):
        pltpu.async_copy(kt_hbm.at[ik_v.at[j]], rk[s], sk[s])
        pltpu.async_copy(qvt_hbm.at[iqv_v.at[j]], rqv[s], sq[s])

    def process(j, s, pf_j, pf_cond):
        @pl.when(pf_cond)
        def _pf():
            start_gather(pf_j, 1 - s)

        pltpu.make_async_copy(kt_hbm.at[ik_v.at[j]], rk[s], sk[s]).wait()
        pltpu.make_async_copy(qvt_hbm.at[iqv_v.at[j]], rqv[s], sq[s]).wait()

        @pl.when(j >= 2)
        def _drain():
            pltpu.make_async_copy(msg[s], acc.at[id_v.at[j]], ss[s]).wait()

        himask = jnp.full((16,), -65536, jnp.int32)

        def unpk(lanes):
            lof = lax.bitcast_convert_type(lanes << 16, jnp.float32)
            hif = lax.bitcast_convert_type(lanes & himask, jnp.float32)
            return lof, hif

        @plsc.parallel_loop(0, C, unroll=4)
        def _erow(e):
            for g in range(D // 32):
                ka = rk[s][e, pl.ds(g * 16, 16)]
                kb = rk[s][e, pl.ds(D // 2 + g * 16, 16)]
                q1, v1 = unpk(rqv[s][e, pl.ds(g * 16, 16)])
                q2, v2 = unpk(rqv[s][e, pl.ds(D // 2 + g * 16, 16)])
                s1 = 1.0 / (1.0 + jnp.exp(-(ka + q1)))
                s2 = 1.0 / (1.0 + jnp.exp(-(kb + q2)))
                msg[s][e, pl.ds(g * 16, 16)] = s1 * v1
                msg[s][e, pl.ds(D // 2 + g * 16, 16)] = s2 * v2

        pltpu.async_copy(msg[s], acc.at[id_v.at[j]], ss[s], add=True)

    nblk = jnp.where(wid == NW - 1, NBLK_L, NBLK)
    ebase = wid * EPW

    @pl.loop(0, nblk)
    def _blk(bi):
        eoff = ebase + bi * BE
        pltpu.sync_copy(ei_hbm.at[0, pl.ds(eoff, BE)], src_v)
        pltpu.sync_copy(ei_hbm.at[1, pl.ds(eoff, BE)], dst_v)
        pltpu.sync_copy(et_hbm.at[pl.ds(eoff, BE)], typ_v)

        @plsc.parallel_loop(0, IB, unroll=2)
        def _idx(j):
            for g in range(C // 16):
                sl = pl.ds(j * C + g * 16, 16)
                gl = pl.ds(g * 16, 16)
                tn = typ_v[sl] * N
                ik_v[j, gl] = tn + dst_v[sl]
                iqv_v[j, gl] = tn + src_v[sl]
                id_v[j, gl] = dst_v[sl]

        start_gather(0, 0)

        @pl.loop(0, IB, step=2)
        def _pair(j):
            process(j, 0, j + 1, True)
            process(j + 1, 1, j + 2, j + 2 < IB)

        pltpu.make_async_copy(msg0, acc.at[id_v.at[IB - 2]], ss0).wait()
        pltpu.make_async_copy(msg1, acc.at[id_v.at[IB - 1]], ss1).wait()

    plsc.subcore_barrier()

    base = cid * N + lo
    pltpu.sync_copy(acc.at[pl.ds(lo, LAST)], out_hbm.at[pl.ds(base, LAST)])

    @pl.when(sid != NS - 1)
    def _tail():
        pltpu.sync_copy(acc.at[pl.ds(lo + LAST, STRIPE - LAST)],
                        out_hbm.at[pl.ds(base + LAST, STRIPE - LAST)])


def _edge_stage(k_tab, qv_tab, edge_index, edge_type):
    mesh = plsc.VectorSubcoreMesh(core_axis_name="c", subcore_axis_name="s")
    run = pl.kernel(
        _edge_body,
        out_type=jax.ShapeDtypeStruct((NC * N, D), jnp.float32),
        mesh=mesh,
        scratch_types=[
            pltpu.VMEM((BE,), jnp.int32),
            pltpu.VMEM((BE,), jnp.int32),
            pltpu.VMEM((BE,), jnp.int32),
            pltpu.VMEM((IB, C), jnp.int32),
            pltpu.VMEM((IB, C), jnp.int32),
            pltpu.VMEM((IB, C), jnp.int32),
            pltpu.VMEM((C, D), jnp.float32),
            pltpu.VMEM((C, D), jnp.float32),
            pltpu.VMEM((C, D), jnp.int32),
            pltpu.VMEM((C, D), jnp.int32),
            pltpu.VMEM((C, D), jnp.float32),
            pltpu.VMEM((C, D), jnp.float32),
            pltpu.VMEM_SHARED((N_ACC, D), jnp.float32),
            pltpu.SemaphoreType.DMA,
            pltpu.SemaphoreType.DMA,
            pltpu.SemaphoreType.DMA,
            pltpu.SemaphoreType.DMA,
            pltpu.SemaphoreType.DMA,
            pltpu.SemaphoreType.DMA,
        ],
    )
    return run(k_tab, qv_tab, edge_index, edge_type)


def _combine_body(p0_ref, p1_ref, skip_ref, out_ref):
    out_ref[...] = (p0_ref[...] + p1_ref[...]) * (1.0 / NE) + skip_ref[...]


def _combine(partial, skip):
    bn = 1000
    return pl.pallas_call(
        _combine_body,
        grid=(N // bn,),
        in_specs=[
            pl.BlockSpec((bn, D), lambda i: (i, 0)),
            pl.BlockSpec((bn, D), lambda i: (i + N // bn, 0)),
            pl.BlockSpec((bn, D), lambda i: (i, 0)),
        ],
        out_specs=pl.BlockSpec((bn, D), lambda i: (i, 0)),
        out_shape=jax.ShapeDtypeStruct((N, D), jnp.float32),
    )(partial, partial, skip)


def kernel(x, edge_index, edge_type, Wk, bk, Wq, bq, Wv, bv, Ws, b):
    assert x.shape == (N, D) and edge_index.shape == (2, E)

    k3, qv3, skip = _make_tables(x, Wk, bk, Wq, bq, Wv, bv, Ws, b)
    k_tab = k3.reshape(NE * N, D)
    qv_tab = qv3.reshape(NE * N, D)

    partial = _edge_stage(k_tab, qv_tab, edge_index, edge_type)
    return _combine(partial, skip)
